# baseline (speedup 1.0000x reference)
import jax
import jax.numpy as jnp
from jax.experimental import pallas as pl
from jax.experimental.pallas import tpu as pltpu

D_MODEL = 2048
NUM_EXPERTS = 64
TOP_K = 2
BT = 1024
NS = 2


def _router_block(x, wt, b):
    logits = jax.lax.dot_general(
        x, wt, (((1,), (1,)), ((), ())),
        preferred_element_type=jnp.float32,
        precision=jax.lax.Precision.DEFAULT,
    )
    logits = logits + b

    iota = jax.lax.broadcasted_iota(jnp.int32, logits.shape, 1)
    m1 = jnp.max(logits, axis=-1, keepdims=True)
    i1 = jnp.min(jnp.where(logits == m1, iota, NUM_EXPERTS), axis=-1, keepdims=True)
    masked = jnp.where(iota == i1, -jnp.inf, logits)
    m2 = jnp.max(masked, axis=-1, keepdims=True)
    i2 = jnp.min(jnp.where(masked == m2, iota, NUM_EXPERTS), axis=-1, keepdims=True)

    e = jnp.exp(logits - m1)
    s = jnp.sum(e, axis=-1, keepdims=True)
    probs = e / s

    p1 = 1.0 / s
    p2 = jnp.exp(m2 - m1) / s
    denom = p1 + p2 + 1e-9
    lane2 = jax.lax.broadcasted_iota(jnp.int32, (x.shape[0], TOP_K), 1)
    tp = jnp.where(lane2 == 0, p1, p2) / denom
    ti = jnp.where(lane2 == 0, i1, i2)
    return probs, tp, ti


def _router_kernel(x1_ref, x2_ref, wt_ref, b_ref, probs_ref, tp_ref, ti_ref):
    wt = wt_ref[...]
    b = b_ref[...]
    for s, x_ref in enumerate((x1_ref, x2_ref)):
        probs, tp, ti = _router_block(x_ref[...], wt, b)
        lo = s * BT
        probs_ref[lo:lo + BT, :] = probs
        tp_ref[lo:lo + BT, :] = tp
        ti_ref[lo:lo + BT, :] = ti


def kernel(x, W, b):
    tokens = x.shape[0]
    b2 = b.reshape(1, NUM_EXPERTS)
    rows_per_step = NS * BT
    grid = (tokens // rows_per_step,)
    probs, topk_probs, topk_idx = pl.pallas_call(
        _router_kernel,
        grid=grid,
        in_specs=[
            pl.BlockSpec((BT, D_MODEL), lambda i: (NS * i, 0)),
            pl.BlockSpec((BT, D_MODEL), lambda i: (NS * i + 1, 0)),
            pl.BlockSpec((NUM_EXPERTS, D_MODEL), lambda i: (0, 0)),
            pl.BlockSpec((1, NUM_EXPERTS), lambda i: (0, 0)),
        ],
        out_specs=[
            pl.BlockSpec((rows_per_step, NUM_EXPERTS), lambda i: (i, 0)),
            pl.BlockSpec((rows_per_step, TOP_K), lambda i: (i, 0)),
            pl.BlockSpec((rows_per_step, TOP_K), lambda i: (i, 0)),
        ],
        out_shape=[
            jax.ShapeDtypeStruct((tokens, NUM_EXPERTS), jnp.float32),
            jax.ShapeDtypeStruct((tokens, TOP_K), jnp.float32),
            jax.ShapeDtypeStruct((tokens, TOP_K), jnp.int32),
        ],
        compiler_params=pltpu.CompilerParams(
            dimension_semantics=("parallel",),
        ),
    )(x, x, W, b2)
    return (probs, topk_probs, topk_idx)


# speedup vs baseline: 1.0010x; 1.0010x over previous
import jax
import jax.numpy as jnp
from jax.experimental import pallas as pl
from jax.experimental.pallas import tpu as pltpu

D_MODEL = 2048
NUM_EXPERTS = 64
TOP_K = 2
BT = 1024
NS = 2


def _router_block(x, wt, b):
    logits = jax.lax.dot_general(
        x, wt, (((1,), (1,)), ((), ())),
        preferred_element_type=jnp.float32,
        precision=jax.lax.Precision.DEFAULT,
    )
    logits = logits + b

    iota = jax.lax.broadcasted_iota(jnp.int32, logits.shape, 1)
    m1 = jnp.max(logits, axis=-1, keepdims=True)
    i1 = jnp.min(jnp.where(logits == m1, iota, NUM_EXPERTS), axis=-1, keepdims=True)
    masked = jnp.where(iota == i1, -jnp.inf, logits)
    m2 = jnp.max(masked, axis=-1, keepdims=True)
    i2 = jnp.min(jnp.where(masked == m2, iota, NUM_EXPERTS), axis=-1, keepdims=True)

    e = jnp.exp(logits - m1)
    s = jnp.sum(e, axis=-1, keepdims=True)
    rs = 1.0 / s
    probs = e * rs

    p1 = rs
    p2 = jnp.exp(m2 - m1) * rs
    denom = p1 + p2 + 1e-9
    lane2 = jax.lax.broadcasted_iota(jnp.int32, (x.shape[0], TOP_K), 1)
    tp = jnp.where(lane2 == 0, p1, p2) / denom
    ti = jnp.where(lane2 == 0, i1, i2)
    return probs, tp, ti


def _router_kernel(x1_ref, x2_ref, wt_ref, b_ref, probs_ref, tp_ref, ti_ref):
    wt = wt_ref[...]
    b = b_ref[...]
    for s, x_ref in enumerate((x1_ref, x2_ref)):
        probs, tp, ti = _router_block(x_ref[...], wt, b)
        lo = s * BT
        probs_ref[lo:lo + BT, :] = probs
        tp_ref[lo:lo + BT, :] = tp
        ti_ref[lo:lo + BT, :] = ti


def kernel(x, W, b):
    tokens = x.shape[0]
    b2 = b.reshape(1, NUM_EXPERTS)
    rows_per_step = NS * BT
    grid = (tokens // rows_per_step,)
    probs, topk_probs, topk_idx = pl.pallas_call(
        _router_kernel,
        grid=grid,
        in_specs=[
            pl.BlockSpec((BT, D_MODEL), lambda i: (NS * i, 0)),
            pl.BlockSpec((BT, D_MODEL), lambda i: (NS * i + 1, 0)),
            pl.BlockSpec((NUM_EXPERTS, D_MODEL), lambda i: (0, 0)),
            pl.BlockSpec((1, NUM_EXPERTS), lambda i: (0, 0)),
        ],
        out_specs=[
            pl.BlockSpec((rows_per_step, NUM_EXPERTS), lambda i: (i, 0)),
            pl.BlockSpec((rows_per_step, TOP_K), lambda i: (i, 0)),
            pl.BlockSpec((rows_per_step, TOP_K), lambda i: (i, 0)),
        ],
        out_shape=[
            jax.ShapeDtypeStruct((tokens, NUM_EXPERTS), jnp.float32),
            jax.ShapeDtypeStruct((tokens, TOP_K), jnp.float32),
            jax.ShapeDtypeStruct((tokens, TOP_K), jnp.int32),
        ],
        compiler_params=pltpu.CompilerParams(
            dimension_semantics=("parallel",),
        ),
    )(x, x, W, b2)
    return (probs, topk_probs, topk_idx)
